# Initial kernel scaffold; baseline (speedup 1.0000x reference)
import jax
import jax.numpy as jnp
from jax.experimental import pallas as pl


def kernel(inputs, gate_w, wi_0, wi_1, wo):
    raise NotImplementedError("write your pallas kernel here")


# speedup vs baseline: 1.2769x; 1.2769x over previous
import functools

import jax
import jax.numpy as jnp
from jax.experimental import pallas as pl
from jax.experimental.pallas import tpu as pltpu

NUM_EXPERTS = 8
TOP_K = 2
SEQ = 2048
D_MODEL = 1024
MLP_DIM = 4096

BT = 256
FB = 1024
NF = MLP_DIM // FB
NB = (SEQ * TOP_K) // BT + NUM_EXPERTS
R = NB * BT


def _moe_mlp_kernel(be_ref, x_ref, w0_ref, w1_ref, wo_ref, s_ref, o_ref):
    j = pl.program_id(1)
    x = x_ref[...]
    h0 = jnp.dot(x, w0_ref[0], preferred_element_type=jnp.float32)
    h1 = jnp.dot(x, w1_ref[0], preferred_element_type=jnp.float32)
    h = jax.nn.silu(h0) * h1
    y = jnp.dot(h, wo_ref[0], preferred_element_type=jnp.float32)

    @pl.when(j == 0)
    def _():
        o_ref[...] = jnp.zeros_like(o_ref)

    o_ref[...] += y

    @pl.when(j == NF - 1)
    def _():
        o_ref[...] = o_ref[...] * s_ref[0, 0][:, None]


@functools.partial(jax.jit, static_argnames=())
def _grouped_mlp(block_expert, x_g, wi_0, wi_1, wo, scale):
    grid_spec = pltpu.PrefetchScalarGridSpec(
        num_scalar_prefetch=1,
        grid=(NB, NF),
        in_specs=[
            pl.BlockSpec((BT, D_MODEL), lambda i, j, be: (i, 0)),
            pl.BlockSpec((1, D_MODEL, FB), lambda i, j, be: (be[i], 0, j)),
            pl.BlockSpec((1, D_MODEL, FB), lambda i, j, be: (be[i], 0, j)),
            pl.BlockSpec((1, FB, D_MODEL), lambda i, j, be: (be[i], j, 0)),
            pl.BlockSpec((1, 1, BT), lambda i, j, be: (i, 0, 0)),
        ],
        out_specs=pl.BlockSpec((BT, D_MODEL), lambda i, j, be: (i, 0)),
    )
    return pl.pallas_call(
        _moe_mlp_kernel,
        grid_spec=grid_spec,
        out_shape=jax.ShapeDtypeStruct((R, D_MODEL), jnp.float32),
        compiler_params=pltpu.CompilerParams(
            dimension_semantics=("arbitrary", "arbitrary"),
        ),
    )(block_expert, x_g, wi_0, wi_1, wo, scale)


def kernel(inputs, gate_w, wi_0, wi_1, wo):
    x = inputs.reshape(SEQ, D_MODEL)

    logits = x @ gate_w
    top_w, sel = jax.lax.top_k(logits, TOP_K)
    top_w = jax.nn.softmax(top_w.astype(jnp.float32), axis=-1)
    experts_flat = sel.reshape(-1)
    weights_flat = top_w.reshape(-1)

    onehot = (experts_flat[:, None] ==
              jnp.arange(NUM_EXPERTS)[None, :]).astype(jnp.int32)
    csum = jnp.cumsum(onehot, axis=0)
    counts = csum[-1]
    ranks = jnp.take_along_axis(csum, experts_flat[:, None], axis=1)[:, 0] - 1
    padded_counts = ((counts + BT - 1) // BT) * BT
    padded_offsets = jnp.concatenate(
        [jnp.zeros((1,), jnp.int32), jnp.cumsum(padded_counts)[:-1]]
    ).astype(jnp.int32)
    pos = padded_offsets[experts_flat] + ranks

    token_of = jnp.arange(SEQ * TOP_K, dtype=jnp.int32) // TOP_K
    gather_idx = jnp.zeros((R,), jnp.int32).at[pos].set(token_of)
    scale_g = jnp.zeros((R,), jnp.float32).at[pos].set(weights_flat)
    block_expert = (
        jnp.searchsorted(padded_offsets,
                         jnp.arange(NB, dtype=jnp.int32) * BT, side="right")
        - 1
    ).astype(jnp.int32)

    x_g = x[gather_idx]

    y_g = _grouped_mlp(block_expert, x_g,
                       wi_0, wi_1, wo,
                       scale_g.reshape(NB, 1, BT))

    out = y_g[pos.reshape(SEQ, TOP_K)].sum(axis=1)
    return out.reshape(1, SEQ, D_MODEL)
